# baseline (speedup 1.0000x reference)
import functools

import jax
import jax.numpy as jnp
from jax import lax
from jax.experimental import pallas as pl
from jax.experimental.pallas import tpu as pltpu
from jax.experimental.pallas import tpu_sc as plsc

_POS_LEN = 19
_NUM_HIST = 5
_HW = 361
_CH0 = 9
_PAD_IDX = (_POS_LEN + 1) * (_POS_LEN + 1) - 1
_BN = 512


def _argmax_body(x_hbm, comb_ref, buf, sem):
    i = pl.program_id(0)
    nb = pl.num_programs(0)

    def copies(blk, slot):
        return [
            pltpu.make_async_copy(
                x_hbm.at[pl.ds(blk * _BN, _BN), _CH0 + c, :],
                buf.at[slot, c], sem.at[slot])
            for c in range(_NUM_HIST)
        ]

    @pl.when(i == 0)
    def _():
        for cp in copies(0, 0):
            cp.start()

    @pl.when(i + 1 < nb)
    def _():
        for cp in copies(i + 1, (i + 1) % 2):
            cp.start()

    for cp in copies(i, i % 2):
        cp.wait()

    flat = buf[i % 2]
    s = jnp.sum(flat, axis=-1)
    m = jnp.max(flat, axis=-1)
    iota = lax.broadcasted_iota(jnp.int32, flat.shape, 2)
    idx = jnp.min(jnp.where(flat == m[..., None], iota, jnp.int32(1 << 20)),
                  axis=-1)
    has = s > 0.5
    rows = (idx * 27) >> 9
    cols = idx - rows * 19
    comb_ref[...] = jnp.where(has, rows * (_POS_LEN + 1) + cols, _PAD_IDX)


def _extract_indices(x):
    n = x.shape[0]
    return pl.pallas_call(
        _argmax_body,
        grid=(n // _BN,),
        in_specs=[pl.BlockSpec(memory_space=pl.ANY)],
        out_specs=pl.BlockSpec((_NUM_HIST, _BN), lambda i: (0, i)),
        out_shape=jax.ShapeDtypeStruct((_NUM_HIST, n), jnp.int32),
        scratch_shapes=[
            pltpu.VMEM((2, _NUM_HIST, _BN, _HW), jnp.float32),
            pltpu.SemaphoreType.DMA((2,)),
        ],
    )(x)


def _sc_gather(table, idx):
    info = plsc.get_sparse_core_info()
    nw = info.num_cores * info.num_subcores
    n = idx.shape[0]
    rpw = n // nw
    cpw = rpw // 128
    d = table.shape[1]
    mesh = plsc.VectorSubcoreMesh(core_axis_name="c", subcore_axis_name="s")

    @functools.partial(
        pl.kernel,
        mesh=mesh,
        out_type=jax.ShapeDtypeStruct((n, d), jnp.float32),
        scratch_types=[
            pltpu.VMEM((rpw,), jnp.int32),
            pltpu.VMEM((rpw, d), jnp.float32),
            pltpu.SemaphoreType.DMA,
            pltpu.SemaphoreType.DMA,
        ],
    )
    def gather_kernel(table_hbm, idx_hbm, out_hbm, idx_v, rows_v, gsem, wsem):
        wid = lax.axis_index("s") * info.num_cores + lax.axis_index("c")
        base = wid * rpw
        pltpu.sync_copy(idx_hbm.at[pl.ds(base, rpw)], idx_v)
        gathers = [
            pltpu.async_copy(table_hbm.at[idx_v.at[pl.ds(j * 128, 128)]],
                             rows_v.at[pl.ds(j * 128, 128)], gsem)
            for j in range(cpw)
        ]
        writes = []
        for j in range(cpw):
            gathers[j].wait()
            writes.append(pltpu.async_copy(
                rows_v.at[pl.ds(j * 128, 128)],
                out_hbm.at[pl.ds(base + j * 128, 128)], wsem))
        for w in writes:
            w.wait()

    return gather_kernel(table, idx)


def _mlp_body(e_ref, w1_ref, b1_ref, w2_ref, b2_ref, o_ref):
    h = b1_ref[...]
    acc = None
    for k in range(_NUM_HIST):
        part = lax.dot_general(
            e_ref[k], w1_ref[:, pl.ds(k * 128, 128)],
            (((1,), (1,)), ((), ())), preferred_element_type=jnp.float32)
        acc = part if acc is None else acc + part
    h = jnp.maximum(acc + h, 0.0)
    o_ref[...] = lax.dot_general(
        h, w2_ref[...], (((1,), (1,)), ((), ())),
        preferred_element_type=jnp.float32) + b2_ref[...]


def _mlp(e3, w1, b1, w2, b2):
    n = e3.shape[1]
    hidden = w1.shape[0]
    c_out = w2.shape[0]
    return pl.pallas_call(
        _mlp_body,
        grid=(n // _BN,),
        in_specs=[
            pl.BlockSpec((_NUM_HIST, _BN, 128), lambda i: (0, i, 0)),
            pl.BlockSpec((hidden, _NUM_HIST * 128), lambda i: (0, 0)),
            pl.BlockSpec((1, hidden), lambda i: (0, 0)),
            pl.BlockSpec((c_out, hidden), lambda i: (0, 0)),
            pl.BlockSpec((1, c_out), lambda i: (0, 0)),
        ],
        out_specs=pl.BlockSpec((_BN, c_out), lambda i: (i, 0)),
        out_shape=jax.ShapeDtypeStruct((n, c_out), jnp.float32),
    )(e3, w1, b1, w2, b2)


def kernel(input_spatial, trunk_out, row_embed, col_embed, W1, b1, W2, b2):
    n = input_spatial.shape[0]
    x = input_spatial.reshape(n, input_spatial.shape[1], _HW)
    comb = _extract_indices(x).reshape(-1)
    table = jnp.concatenate(
        [jnp.repeat(row_embed, _POS_LEN + 1, axis=0),
         jnp.tile(col_embed, (_POS_LEN + 1, 1))], axis=-1)
    emb = _sc_gather(table, comb)
    out = _mlp(emb.reshape(_NUM_HIST, n, 128),
               W1, b1.reshape(1, -1), W2, b2.reshape(1, -1))
    return out[:, :, None, None]
